# Initial kernel scaffold; baseline (speedup 1.0000x reference)
import jax
import jax.numpy as jnp
from jax.experimental import pallas as pl


def kernel(x, edge_index, W_est, b_est, fc_w, fc_b, W_gnn, b_gnn, cls_w, cls_b):
    raise NotImplementedError("write your pallas kernel here")


# speedup vs baseline: 35.9027x; 35.9027x over previous
import functools

import jax
import jax.numpy as jnp
from jax import lax
from jax.experimental import pallas as pl
from jax.experimental.pallas import tpu as pltpu
from jax.experimental.pallas import tpu_sc as plsc

N = 10000
E = 320000
F = 128
NC = 2
NS = 16
NW = NC * NS
CH = 128
KC = 80
EPT = CH * KC
EPAD = NW * EPT
NPAD = 10240
NP2 = 2 * NPAD
RPS = NP2 // NS
L = 16

_mesh = plsc.VectorSubcoreMesh(core_axis_name="c", subcore_axis_name="s",
                               num_cores=NC, num_subcores=NS)
_sc_params = pltpu.CompilerParams(use_tc_tiling_on_sc=False)


def _fill(ref, n, value):
    vec = jnp.full((L,), value, jnp.float32)
    for k in range(n // L):
        ref[pl.ds(k * L, L)] = vec


@functools.partial(
    pl.kernel,
    out_type=jax.ShapeDtypeStruct((NC * NP2,), jnp.float32),
    mesh=_mesh,
    compiler_params=_sc_params,
    scratch_types=[
        pltpu.VMEM((KC, CH), jnp.int32),
        pltpu.VMEM((KC, CH), jnp.int32),
        pltpu.VMEM((CH,), jnp.int32),
        pltpu.VMEM((CH,), jnp.int32),
        pltpu.VMEM((CH,), jnp.float32),
        pltpu.VMEM((RPS,), jnp.float32),
        pltpu.VMEM_SHARED((NP2,), jnp.float32),
    ],
)
def _deg_kernel(src_hbm, dst_hbm, degp_hbm,
                idx_s, idx_d, cur_s, cur_d, ones_v, zero_v, deg_sh):
    c = lax.axis_index("c")
    s = lax.axis_index("s")
    wid = c * NS + s
    _fill(ones_v, CH, 1.0)
    _fill(zero_v, RPS, 0.0)
    pltpu.sync_copy(zero_v, deg_sh.at[pl.ds(s * RPS, RPS)])
    pltpu.sync_copy(src_hbm.at[wid], idx_s)
    pltpu.sync_copy(dst_hbm.at[wid], idx_d)
    plsc.subcore_barrier()

    def body(j, carry):
        for k in range(CH // L):
            cur_s[pl.ds(k * L, L)] = idx_s[j, pl.ds(k * L, L)]
            cur_d[pl.ds(k * L, L)] = idx_d[j, pl.ds(k * L, L)] + NPAD
        pltpu.sync_copy(ones_v, deg_sh.at[cur_s], add=True)
        pltpu.sync_copy(ones_v, deg_sh.at[cur_d], add=True)
        return carry

    lax.fori_loop(0, KC, body, 0)
    plsc.subcore_barrier()
    pltpu.sync_copy(deg_sh.at[pl.ds(s * RPS, RPS)],
                    degp_hbm.at[pl.ds(c * NP2 + s * RPS, RPS)])


@functools.partial(
    pl.kernel,
    out_type=jax.ShapeDtypeStruct((NC * NP2,), jnp.float32),
    mesh=_mesh,
    compiler_params=_sc_params,
    scratch_types=[
        pltpu.VMEM((KC, CH), jnp.int32),
        pltpu.VMEM((KC, CH), jnp.int32),
        pltpu.VMEM((CH,), jnp.int32),
        pltpu.VMEM((CH,), jnp.int32),
        pltpu.VMEM((CH,), jnp.int32),
        pltpu.VMEM((CH,), jnp.int32),
        pltpu.VMEM((CH,), jnp.float32),
        pltpu.VMEM((CH,), jnp.float32),
        pltpu.VMEM((RPS,), jnp.float32),
        pltpu.SemaphoreType.DMA,
        pltpu.SemaphoreType.DMA,
        pltpu.VMEM_SHARED((NP2,), jnp.float32),
    ],
)
def _agg_kernel(src_hbm, dst_hbm, v_hbm, aggp_hbm,
                idx_s, idx_d, cur_s0, cur_s1, cur_d0, cur_d1,
                msg0, msg1, zero_v, sem0, sem1, agg_sh):
    c = lax.axis_index("c")
    s = lax.axis_index("s")
    wid = c * NS + s
    _fill(zero_v, RPS, 0.0)
    pltpu.sync_copy(zero_v, agg_sh.at[pl.ds(s * RPS, RPS)])
    pltpu.sync_copy(src_hbm.at[wid], idx_s)
    pltpu.sync_copy(dst_hbm.at[wid], idx_d)
    plsc.subcore_barrier()

    def body(j, carry):
        for k in range(CH // L):
            sv = idx_s[j, pl.ds(k * L, L)]
            dv = idx_d[j, pl.ds(k * L, L)]
            cur_s0[pl.ds(k * L, L)] = sv
            cur_s1[pl.ds(k * L, L)] = sv + NPAD
            cur_d0[pl.ds(k * L, L)] = dv
            cur_d1[pl.ds(k * L, L)] = dv + NPAD
        cp0 = pltpu.async_copy(v_hbm.at[cur_s0], msg0, sem0)
        cp1 = pltpu.async_copy(v_hbm.at[cur_s1], msg1, sem1)
        cp0.wait()
        cp1.wait()
        pltpu.sync_copy(msg0, agg_sh.at[cur_d0], add=True)
        pltpu.sync_copy(msg1, agg_sh.at[cur_d1], add=True)
        return carry

    lax.fori_loop(0, KC, body, 0)
    plsc.subcore_barrier()
    pltpu.sync_copy(agg_sh.at[pl.ds(s * RPS, RPS)],
                    aggp_hbm.at[pl.ds(c * NP2 + s * RPS, RPS)])


def _mid_body(x_ref, we_ref, fw_ref, wg_ref, cw_ref, deg4_ref, v_ref):
    wc1 = jnp.dot(we_ref[...], fw_ref[...], preferred_element_type=jnp.float32)
    wc2 = jnp.dot(wg_ref[...], cw_ref[...], preferred_element_type=jnp.float32)
    u1 = jnp.dot(x_ref[...], wc1, preferred_element_type=jnp.float32)
    u2 = jnp.dot(x_ref[...], wc2, preferred_element_type=jnp.float32)
    dout = deg4_ref[0:1, :] + deg4_ref[2:3, :]
    ns_row = jnp.where(dout > 0, lax.rsqrt(jnp.maximum(dout, 1.0)), 0.0)
    ns = jnp.transpose(ns_row)
    v_ref[...] = jnp.concatenate(
        [jnp.transpose(u1 * ns), jnp.transpose(u2 * ns)], axis=0)


def _fin_body(agg4_ref, deg4_ref, be_ref, fw_ref, fb_ref, bg_ref, cw_ref,
              cb_ref, s_ref, y_ref):
    din = deg4_ref[1:2, :] + deg4_ref[3:4, :]
    nd = jnp.where(din > 0, lax.rsqrt(jnp.maximum(din, 1.0)), 0.0)
    agg0 = agg4_ref[0:1, :] + agg4_ref[2:3, :]
    agg1 = agg4_ref[1:2, :] + agg4_ref[3:4, :]
    c1 = jnp.sum(be_ref[...] * fw_ref[...]) + fb_ref[0, 0]
    c2 = jnp.sum(bg_ref[...] * cw_ref[...]) + cb_ref[0, 0]
    s_ref[...] = agg0 * nd + c1
    y_ref[...] = agg1 * nd + c2


_mid_call = pl.pallas_call(
    _mid_body,
    out_shape=jax.ShapeDtypeStruct((2, NPAD), jnp.float32),
)

_fin_call = pl.pallas_call(
    _fin_body,
    out_shape=[
        jax.ShapeDtypeStruct((1, NPAD), jnp.float32),
        jax.ShapeDtypeStruct((1, NPAD), jnp.float32),
    ],
)


def kernel(x, edge_index, W_est, b_est, fc_w, fc_b, W_gnn, b_gnn, cls_w, cls_b):
    src = edge_index[0]
    dst = edge_index[1]
    pad = EPAD - E
    pad_idx = (N + (jnp.arange(pad, dtype=jnp.int32) % 32)).astype(jnp.int32)
    src_t = jnp.concatenate([src, pad_idx]).reshape(NW, KC, CH)
    dst_t = jnp.concatenate([dst, pad_idx]).reshape(NW, KC, CH)

    x_pad = jnp.pad(x, ((0, NPAD - N), (0, 0)))

    degp = _deg_kernel(src_t, dst_t)
    deg4 = degp.reshape(NC * 2, NPAD)
    v01 = _mid_call(x_pad, W_est, fc_w, W_gnn, cls_w, deg4)
    aggp = _agg_kernel(src_t, dst_t, v01.reshape(NP2), )
    agg4 = aggp.reshape(NC * 2, NPAD)
    s_row, y_row = _fin_call(
        agg4, deg4,
        b_est.reshape(1, F), fc_w.reshape(1, F), fc_b.reshape(1, 1),
        b_gnn.reshape(1, F), cls_w.reshape(1, F), cls_b.reshape(1, 1),
    )
    return (y_row.reshape(NPAD, 1)[:N], s_row.reshape(NPAD, 1)[:N])


# speedup vs baseline: 49.0470x; 1.3661x over previous
import functools

import jax
import jax.numpy as jnp
from jax import lax
from jax.experimental import pallas as pl
from jax.experimental.pallas import tpu as pltpu
from jax.experimental.pallas import tpu_sc as plsc

N = 10000
E = 320000
F = 128
NC = 2
NS = 16
NW = NC * NS
CH = 128
KC = 80
EPT = CH * KC
EPAD = NW * EPT
NPAD = 10240
NP2 = 2 * NPAD
RPS = NP2 // NS
L = 16

_mesh = plsc.VectorSubcoreMesh(core_axis_name="c", subcore_axis_name="s",
                               num_cores=NC, num_subcores=NS)
_sc_params = pltpu.CompilerParams(use_tc_tiling_on_sc=False)


def _fill(ref, n, value):
    vec = jnp.full((L,), value, jnp.float32)
    for k in range(n // L):
        ref[pl.ds(k * L, L)] = vec


@functools.partial(
    pl.kernel,
    out_type=jax.ShapeDtypeStruct((NC * NP2,), jnp.float32),
    mesh=_mesh,
    compiler_params=_sc_params,
    scratch_types=[
        pltpu.VMEM((KC, CH), jnp.int32),
        pltpu.VMEM((KC, CH), jnp.int32),
        pltpu.VMEM((CH,), jnp.float32),
        pltpu.VMEM((RPS,), jnp.float32),
        pltpu.SemaphoreType.DMA,
        pltpu.VMEM_SHARED((NP2,), jnp.float32),
    ],
)
def _deg_kernel(src0_hbm, dst1_hbm, degp_hbm,
                idx_s, idx_d, ones_v, zero_v, ssem, deg_sh):
    c = lax.axis_index("c")
    s = lax.axis_index("s")
    wid = c * NS + s
    _fill(ones_v, CH, 1.0)
    _fill(zero_v, RPS, 0.0)
    pltpu.sync_copy(zero_v, deg_sh.at[pl.ds(s * RPS, RPS)])
    pltpu.sync_copy(src0_hbm.at[wid], idx_s)
    pltpu.sync_copy(dst1_hbm.at[wid], idx_d)
    plsc.subcore_barrier()

    def fire(j, carry):
        pltpu.async_copy(ones_v, deg_sh.at[idx_s.at[j]], ssem, add=True)
        pltpu.async_copy(ones_v, deg_sh.at[idx_d.at[j]], ssem, add=True)
        return carry

    lax.fori_loop(0, KC, fire, 0)

    def drain(j, carry):
        pltpu.make_async_copy(ones_v, deg_sh.at[idx_s.at[j]], ssem).wait()
        pltpu.make_async_copy(ones_v, deg_sh.at[idx_d.at[j]], ssem).wait()
        return carry

    lax.fori_loop(0, KC, drain, 0)
    plsc.subcore_barrier()
    pltpu.sync_copy(deg_sh.at[pl.ds(s * RPS, RPS)],
                    degp_hbm.at[pl.ds(c * NP2 + s * RPS, RPS)])


@functools.partial(
    pl.kernel,
    out_type=jax.ShapeDtypeStruct((NC * NP2,), jnp.float32),
    mesh=_mesh,
    compiler_params=_sc_params,
    scratch_types=[
        pltpu.VMEM((KC, CH), jnp.int32),
        pltpu.VMEM((KC, CH), jnp.int32),
        pltpu.VMEM((KC, CH), jnp.int32),
        pltpu.VMEM((KC, CH), jnp.int32),
        pltpu.VMEM((KC, CH), jnp.float32),
        pltpu.VMEM((KC, CH), jnp.float32),
        pltpu.VMEM((RPS,), jnp.float32),
        pltpu.SemaphoreType.DMA,
        pltpu.SemaphoreType.DMA,
        pltpu.VMEM_SHARED((NP2,), jnp.float32),
    ],
)
def _agg_kernel(src0_hbm, src1_hbm, dst0_hbm, dst1_hbm, v_hbm, aggp_hbm,
                idx_s0, idx_s1, idx_d0, idx_d1, msg0, msg1, zero_v,
                gsem, ssem, agg_sh):
    c = lax.axis_index("c")
    s = lax.axis_index("s")
    wid = c * NS + s
    _fill(zero_v, RPS, 0.0)
    pltpu.sync_copy(zero_v, agg_sh.at[pl.ds(s * RPS, RPS)])
    pltpu.sync_copy(src0_hbm.at[wid], idx_s0)
    pltpu.sync_copy(src1_hbm.at[wid], idx_s1)
    pltpu.sync_copy(dst0_hbm.at[wid], idx_d0)
    pltpu.sync_copy(dst1_hbm.at[wid], idx_d1)
    plsc.subcore_barrier()

    def gfire(j, carry):
        pltpu.async_copy(v_hbm.at[idx_s0.at[j]], msg0.at[j], gsem)
        pltpu.async_copy(v_hbm.at[idx_s1.at[j]], msg1.at[j], gsem)
        return carry

    lax.fori_loop(0, KC, gfire, 0)

    def gdrain(j, carry):
        pltpu.make_async_copy(v_hbm.at[idx_s0.at[j]], msg0.at[j], gsem).wait()
        pltpu.make_async_copy(v_hbm.at[idx_s1.at[j]], msg1.at[j], gsem).wait()
        return carry

    lax.fori_loop(0, KC, gdrain, 0)

    def sfire(j, carry):
        pltpu.async_copy(msg0.at[j], agg_sh.at[idx_d0.at[j]], ssem, add=True)
        pltpu.async_copy(msg1.at[j], agg_sh.at[idx_d1.at[j]], ssem, add=True)
        return carry

    lax.fori_loop(0, KC, sfire, 0)

    def sdrain(j, carry):
        pltpu.make_async_copy(msg0.at[j], agg_sh.at[idx_d0.at[j]], ssem).wait()
        pltpu.make_async_copy(msg1.at[j], agg_sh.at[idx_d1.at[j]], ssem).wait()
        return carry

    lax.fori_loop(0, KC, sdrain, 0)
    plsc.subcore_barrier()
    pltpu.sync_copy(agg_sh.at[pl.ds(s * RPS, RPS)],
                    aggp_hbm.at[pl.ds(c * NP2 + s * RPS, RPS)])


def _mid_body(x_ref, we_ref, fw_ref, wg_ref, cw_ref, deg4_ref, v_ref):
    wc1 = jnp.dot(we_ref[...], fw_ref[...], preferred_element_type=jnp.float32)
    wc2 = jnp.dot(wg_ref[...], cw_ref[...], preferred_element_type=jnp.float32)
    u1 = jnp.dot(x_ref[...], wc1, preferred_element_type=jnp.float32)
    u2 = jnp.dot(x_ref[...], wc2, preferred_element_type=jnp.float32)
    dout = deg4_ref[0:1, :] + deg4_ref[2:3, :]
    ns_row = jnp.where(dout > 0, lax.rsqrt(jnp.maximum(dout, 1.0)), 0.0)
    ns = jnp.transpose(ns_row)
    v_ref[...] = jnp.concatenate(
        [jnp.transpose(u1 * ns), jnp.transpose(u2 * ns)], axis=0)


def _fin_body(agg4_ref, deg4_ref, be_ref, fw_ref, fb_ref, bg_ref, cw_ref,
              cb_ref, s_ref, y_ref):
    din = deg4_ref[1:2, :] + deg4_ref[3:4, :]
    nd = jnp.where(din > 0, lax.rsqrt(jnp.maximum(din, 1.0)), 0.0)
    agg0 = agg4_ref[0:1, :] + agg4_ref[2:3, :]
    agg1 = agg4_ref[1:2, :] + agg4_ref[3:4, :]
    c1 = jnp.sum(be_ref[...] * fw_ref[...]) + fb_ref[0, 0]
    c2 = jnp.sum(bg_ref[...] * cw_ref[...]) + cb_ref[0, 0]
    s_ref[...] = agg0 * nd + c1
    y_ref[...] = agg1 * nd + c2


_mid_call = pl.pallas_call(
    _mid_body,
    out_shape=jax.ShapeDtypeStruct((2, NPAD), jnp.float32),
)

_fin_call = pl.pallas_call(
    _fin_body,
    out_shape=[
        jax.ShapeDtypeStruct((1, NPAD), jnp.float32),
        jax.ShapeDtypeStruct((1, NPAD), jnp.float32),
    ],
)


def kernel(x, edge_index, W_est, b_est, fc_w, fc_b, W_gnn, b_gnn, cls_w, cls_b):
    src = edge_index[0]
    dst = edge_index[1]
    pad = EPAD - E
    pad_idx = (N + (jnp.arange(pad, dtype=jnp.int32) % 32)).astype(jnp.int32)
    src0 = jnp.concatenate([src, pad_idx]).reshape(NW, KC, CH)
    dst0 = jnp.concatenate([dst, pad_idx]).reshape(NW, KC, CH)
    src1 = src0 + NPAD
    dst1 = dst0 + NPAD

    x_pad = jnp.pad(x, ((0, NPAD - N), (0, 0)))

    degp = _deg_kernel(src0, dst1)
    deg4 = degp.reshape(NC * 2, NPAD)
    v01 = _mid_call(x_pad, W_est, fc_w, W_gnn, cls_w, deg4)
    aggp = _agg_kernel(src0, src1, dst0, dst1, v01.reshape(NP2))
    agg4 = aggp.reshape(NC * 2, NPAD)
    s_row, y_row = _fin_call(
        agg4, deg4,
        b_est.reshape(1, F), fc_w.reshape(1, F), fc_b.reshape(1, 1),
        b_gnn.reshape(1, F), cls_w.reshape(1, F), cls_b.reshape(1, 1),
    )
    return (y_row.reshape(NPAD, 1)[:N], s_row.reshape(NPAD, 1)[:N])


# speedup vs baseline: 72.9843x; 1.4880x over previous
import functools

import jax
import jax.numpy as jnp
from jax import lax
from jax.experimental import pallas as pl
from jax.experimental.pallas import tpu as pltpu
from jax.experimental.pallas import tpu_sc as plsc

N = 10000
E = 320000
F = 128
NC = 2
NS = 16
NW = NC * NS
CH = 128
KC = 80
EPT = CH * KC
EPAD = NW * EPT
NPAD = 10240
NP2 = 2 * NPAD
RPS = NP2 // NS
L = 16

_mesh = plsc.VectorSubcoreMesh(core_axis_name="c", subcore_axis_name="s",
                               num_cores=NC, num_subcores=NS)
_sc_params = pltpu.CompilerParams(use_tc_tiling_on_sc=False)


def _fill(ref, n, value):
    vec = jnp.full((L,), value, jnp.float32)
    for k in range(n // L):
        ref[pl.ds(k * L, L)] = vec


@functools.partial(
    pl.kernel,
    out_type=jax.ShapeDtypeStruct((NC * NP2,), jnp.float32),
    mesh=_mesh,
    compiler_params=_sc_params,
    scratch_types=[
        pltpu.VMEM((KC, CH), jnp.int32),
        pltpu.VMEM((KC, CH), jnp.int32),
        pltpu.VMEM((CH,), jnp.float32),
        pltpu.VMEM((RPS,), jnp.float32),
        pltpu.SemaphoreType.DMA,
        pltpu.VMEM_SHARED((NP2,), jnp.float32),
    ],
)
def _deg_kernel(src0_hbm, dst0_hbm, degp_hbm,
                idx_s, idx_d, ones_v, zero_v, ssem, deg_sh):
    c = lax.axis_index("c")
    s = lax.axis_index("s")
    wid = c * NS + s
    _fill(ones_v, CH, 1.0)
    _fill(zero_v, RPS, 0.0)
    pltpu.sync_copy(zero_v, deg_sh.at[pl.ds(s * RPS, RPS)])
    pltpu.sync_copy(src0_hbm.at[wid], idx_s)
    pltpu.sync_copy(dst0_hbm.at[wid], idx_d)

    def mk(j, carry):
        for k in range(CH // L):
            idx_d[j, pl.ds(k * L, L)] = idx_d[j, pl.ds(k * L, L)] + NPAD
        return carry

    lax.fori_loop(0, KC, mk, 0)
    plsc.subcore_barrier()

    def fire(j, carry):
        pltpu.async_copy(ones_v, deg_sh.at[idx_s.at[j]], ssem, add=True)
        pltpu.async_copy(ones_v, deg_sh.at[idx_d.at[j]], ssem, add=True)
        return carry

    lax.fori_loop(0, KC, fire, 0)

    def drain(j, carry):
        pltpu.make_async_copy(ones_v, deg_sh.at[idx_s.at[j]], ssem).wait()
        pltpu.make_async_copy(ones_v, deg_sh.at[idx_d.at[j]], ssem).wait()
        return carry

    lax.fori_loop(0, KC, drain, 0)
    plsc.subcore_barrier()
    pltpu.sync_copy(deg_sh.at[pl.ds(s * RPS, RPS)],
                    degp_hbm.at[pl.ds(c * NP2 + s * RPS, RPS)])


@functools.partial(
    pl.kernel,
    out_type=jax.ShapeDtypeStruct((NC * NP2,), jnp.float32),
    mesh=_mesh,
    compiler_params=_sc_params,
    scratch_types=[
        pltpu.VMEM((KC, CH), jnp.int32),
        pltpu.VMEM((KC, CH), jnp.int32),
        pltpu.VMEM((KC, CH), jnp.int32),
        pltpu.VMEM((KC, CH), jnp.int32),
        pltpu.VMEM((KC, CH), jnp.float32),
        pltpu.VMEM((KC, CH), jnp.float32),
        pltpu.VMEM((RPS,), jnp.float32),
        pltpu.SemaphoreType.DMA,
        pltpu.SemaphoreType.DMA,
        pltpu.VMEM_SHARED((NP2,), jnp.float32),
        pltpu.VMEM_SHARED((NP2,), jnp.float32),
    ],
)
def _agg_kernel(src0_hbm, dst0_hbm, v_hbm, aggp_hbm,
                idx_s0, idx_s1, idx_d0, idx_d1, msg0, msg1, zero_v,
                gsem, ssem, agg_sh, v_sh):
    c = lax.axis_index("c")
    s = lax.axis_index("s")
    wid = c * NS + s
    _fill(zero_v, RPS, 0.0)
    pltpu.sync_copy(zero_v, agg_sh.at[pl.ds(s * RPS, RPS)])
    pltpu.sync_copy(v_hbm.at[pl.ds(s * RPS, RPS)], v_sh.at[pl.ds(s * RPS, RPS)])
    pltpu.sync_copy(src0_hbm.at[wid], idx_s0)
    pltpu.sync_copy(dst0_hbm.at[wid], idx_d0)

    def mk(j, carry):
        for k in range(CH // L):
            idx_s1[j, pl.ds(k * L, L)] = idx_s0[j, pl.ds(k * L, L)] + NPAD
            idx_d1[j, pl.ds(k * L, L)] = idx_d0[j, pl.ds(k * L, L)] + NPAD
        return carry

    lax.fori_loop(0, KC, mk, 0)
    plsc.subcore_barrier()

    def gfire(j, carry):
        pltpu.async_copy(v_sh.at[idx_s0.at[j]], msg0.at[j], gsem)
        pltpu.async_copy(v_sh.at[idx_s1.at[j]], msg1.at[j], gsem)
        return carry

    lax.fori_loop(0, KC, gfire, 0)

    def pipe(j, carry):
        pltpu.make_async_copy(v_sh.at[idx_s0.at[j]], msg0.at[j], gsem).wait()
        pltpu.make_async_copy(v_sh.at[idx_s1.at[j]], msg1.at[j], gsem).wait()
        pltpu.async_copy(msg0.at[j], agg_sh.at[idx_d0.at[j]], ssem, add=True)
        pltpu.async_copy(msg1.at[j], agg_sh.at[idx_d1.at[j]], ssem, add=True)
        return carry

    lax.fori_loop(0, KC, pipe, 0)

    def sdrain(j, carry):
        pltpu.make_async_copy(msg0.at[j], agg_sh.at[idx_d0.at[j]], ssem).wait()
        pltpu.make_async_copy(msg1.at[j], agg_sh.at[idx_d1.at[j]], ssem).wait()
        return carry

    lax.fori_loop(0, KC, sdrain, 0)
    plsc.subcore_barrier()
    pltpu.sync_copy(agg_sh.at[pl.ds(s * RPS, RPS)],
                    aggp_hbm.at[pl.ds(c * NP2 + s * RPS, RPS)])


def _mid_body(x_ref, we_ref, fw_ref, wg_ref, cw_ref, deg4_ref, v_ref):
    wc1 = jnp.dot(we_ref[...], fw_ref[...], preferred_element_type=jnp.float32)
    wc2 = jnp.dot(wg_ref[...], cw_ref[...], preferred_element_type=jnp.float32)
    u1 = jnp.dot(x_ref[...], wc1, preferred_element_type=jnp.float32)
    u2 = jnp.dot(x_ref[...], wc2, preferred_element_type=jnp.float32)
    dout = deg4_ref[0:1, :] + deg4_ref[2:3, :]
    ns_row = jnp.where(dout > 0, lax.rsqrt(jnp.maximum(dout, 1.0)), 0.0)
    ns = jnp.transpose(ns_row)
    v_ref[...] = jnp.concatenate(
        [jnp.transpose(u1 * ns), jnp.transpose(u2 * ns)], axis=0)


def _fin_body(agg4_ref, deg4_ref, be_ref, fw_ref, fb_ref, bg_ref, cw_ref,
              cb_ref, s_ref, y_ref):
    din = deg4_ref[1:2, :] + deg4_ref[3:4, :]
    nd = jnp.where(din > 0, lax.rsqrt(jnp.maximum(din, 1.0)), 0.0)
    agg0 = agg4_ref[0:1, :] + agg4_ref[2:3, :]
    agg1 = agg4_ref[1:2, :] + agg4_ref[3:4, :]
    c1 = jnp.sum(be_ref[...] * fw_ref[...]) + fb_ref[0, 0]
    c2 = jnp.sum(bg_ref[...] * cw_ref[...]) + cb_ref[0, 0]
    s_ref[...] = agg0 * nd + c1
    y_ref[...] = agg1 * nd + c2


_mid_call = pl.pallas_call(
    _mid_body,
    out_shape=jax.ShapeDtypeStruct((2, NPAD), jnp.float32),
)

_fin_call = pl.pallas_call(
    _fin_body,
    out_shape=[
        jax.ShapeDtypeStruct((1, NPAD), jnp.float32),
        jax.ShapeDtypeStruct((1, NPAD), jnp.float32),
    ],
)


def kernel(x, edge_index, W_est, b_est, fc_w, fc_b, W_gnn, b_gnn, cls_w, cls_b):
    src = edge_index[0]
    dst = edge_index[1]
    pad = EPAD - E
    pad_idx = (N + (jnp.arange(pad, dtype=jnp.int32) % 32)).astype(jnp.int32)
    src0 = jnp.concatenate([src, pad_idx]).reshape(NW, KC, CH)
    dst0 = jnp.concatenate([dst, pad_idx]).reshape(NW, KC, CH)

    x_pad = jnp.pad(x, ((0, NPAD - N), (0, 0)))

    degp = _deg_kernel(src0, dst0)
    deg4 = degp.reshape(NC * 2, NPAD)
    v01 = _mid_call(x_pad, W_est, fc_w, W_gnn, cls_w, deg4)
    aggp = _agg_kernel(src0, dst0, v01.reshape(NP2))
    agg4 = aggp.reshape(NC * 2, NPAD)
    s_row, y_row = _fin_call(
        agg4, deg4,
        b_est.reshape(1, F), fc_w.reshape(1, F), fc_b.reshape(1, 1),
        b_gnn.reshape(1, F), cls_w.reshape(1, F), cls_b.reshape(1, 1),
    )
    return (y_row.reshape(NPAD, 1)[:N], s_row.reshape(NPAD, 1)[:N])


# speedup vs baseline: 79.9676x; 1.0957x over previous
import functools

import jax
import jax.numpy as jnp
from jax import lax
from jax.experimental import pallas as pl
from jax.experimental.pallas import tpu as pltpu
from jax.experimental.pallas import tpu_sc as plsc

N = 10000
E = 320000
F = 128
NC = 2
NS = 16
NW = NC * NS
CH = 128
KC = 80
EPT = CH * KC
EPAD = NW * EPT
NPAD = 10240
NP2 = 2 * NPAD
RPN = NPAD // NS
L = 16

_mesh = plsc.VectorSubcoreMesh(core_axis_name="c", subcore_axis_name="s",
                               num_cores=NC, num_subcores=NS)
_sc_params = pltpu.CompilerParams(use_tc_tiling_on_sc=False)


def _fill(ref, n, value):
    vec = jnp.full((L,), value, jnp.float32)
    for k in range(n // L):
        ref[pl.ds(k * L, L)] = vec


@functools.partial(
    pl.kernel,
    out_type=jax.ShapeDtypeStruct((NC * NP2,), jnp.float32),
    mesh=_mesh,
    compiler_params=_sc_params,
    scratch_types=[
        pltpu.VMEM((KC, CH), jnp.int32),
        pltpu.VMEM((KC, CH), jnp.int32),
        pltpu.VMEM((CH,), jnp.float32),
        pltpu.VMEM((RPN,), jnp.float32),
        pltpu.SemaphoreType.DMA,
        pltpu.VMEM_SHARED((NPAD,), jnp.float32),
        pltpu.VMEM_SHARED((NPAD,), jnp.float32),
    ],
)
def _deg_kernel(src0_hbm, dst0_hbm, degp_hbm,
                idx_s, idx_d, ones_v, zero_v, ssem, dsh_out, dsh_in):
    c = lax.axis_index("c")
    s = lax.axis_index("s")
    wid = c * NS + s
    _fill(ones_v, CH, 1.0)
    _fill(zero_v, RPN, 0.0)
    pltpu.sync_copy(zero_v, dsh_out.at[pl.ds(s * RPN, RPN)])
    pltpu.sync_copy(zero_v, dsh_in.at[pl.ds(s * RPN, RPN)])
    pltpu.sync_copy(src0_hbm.at[wid], idx_s)
    pltpu.sync_copy(dst0_hbm.at[wid], idx_d)
    plsc.subcore_barrier()

    def fire(j, carry):
        pltpu.async_copy(ones_v, dsh_out.at[idx_s.at[j]], ssem, add=True)
        pltpu.async_copy(ones_v, dsh_in.at[idx_d.at[j]], ssem, add=True)
        return carry

    lax.fori_loop(0, KC, fire, 0)

    def drain(j, carry):
        pltpu.make_async_copy(ones_v, dsh_out.at[idx_s.at[j]], ssem).wait()
        pltpu.make_async_copy(ones_v, dsh_in.at[idx_d.at[j]], ssem).wait()
        return carry

    lax.fori_loop(0, KC, drain, 0)
    plsc.subcore_barrier()
    pltpu.sync_copy(dsh_out.at[pl.ds(s * RPN, RPN)],
                    degp_hbm.at[pl.ds(c * NP2 + s * RPN, RPN)])
    pltpu.sync_copy(dsh_in.at[pl.ds(s * RPN, RPN)],
                    degp_hbm.at[pl.ds(c * NP2 + NPAD + s * RPN, RPN)])


@functools.partial(
    pl.kernel,
    out_type=jax.ShapeDtypeStruct((NC * NP2,), jnp.float32),
    mesh=_mesh,
    compiler_params=_sc_params,
    scratch_types=[
        pltpu.VMEM((KC, CH), jnp.int32),
        pltpu.VMEM((KC, CH), jnp.int32),
        pltpu.VMEM((KC, CH), jnp.float32),
        pltpu.VMEM((KC, CH), jnp.float32),
        pltpu.VMEM((RPN,), jnp.float32),
        pltpu.SemaphoreType.DMA,
        pltpu.SemaphoreType.DMA,
        pltpu.VMEM_SHARED((NPAD,), jnp.float32),
        pltpu.VMEM_SHARED((NPAD,), jnp.float32),
        pltpu.VMEM_SHARED((NPAD,), jnp.float32),
        pltpu.VMEM_SHARED((NPAD,), jnp.float32),
    ],
)
def _agg_kernel(src0_hbm, dst0_hbm, v_hbm, aggp_hbm,
                idx_s, idx_d, msg0, msg1, zero_v, gsem, ssem,
                ash0, ash1, vsh0, vsh1):
    c = lax.axis_index("c")
    s = lax.axis_index("s")
    wid = c * NS + s
    _fill(zero_v, RPN, 0.0)
    pltpu.sync_copy(zero_v, ash0.at[pl.ds(s * RPN, RPN)])
    pltpu.sync_copy(zero_v, ash1.at[pl.ds(s * RPN, RPN)])
    pltpu.sync_copy(v_hbm.at[pl.ds(s * RPN, RPN)],
                    vsh0.at[pl.ds(s * RPN, RPN)])
    pltpu.sync_copy(v_hbm.at[pl.ds(NPAD + s * RPN, RPN)],
                    vsh1.at[pl.ds(s * RPN, RPN)])
    pltpu.sync_copy(src0_hbm.at[wid], idx_s)
    pltpu.sync_copy(dst0_hbm.at[wid], idx_d)
    plsc.subcore_barrier()

    def gfire(j, carry):
        pltpu.async_copy(vsh0.at[idx_s.at[j]], msg0.at[j], gsem)
        pltpu.async_copy(vsh1.at[idx_s.at[j]], msg1.at[j], gsem)
        return carry

    lax.fori_loop(0, KC, gfire, 0)

    def pipe(j, carry):
        pltpu.make_async_copy(vsh0.at[idx_s.at[j]], msg0.at[j], gsem).wait()
        pltpu.make_async_copy(vsh1.at[idx_s.at[j]], msg1.at[j], gsem).wait()
        pltpu.async_copy(msg0.at[j], ash0.at[idx_d.at[j]], ssem, add=True)
        pltpu.async_copy(msg1.at[j], ash1.at[idx_d.at[j]], ssem, add=True)
        return carry

    lax.fori_loop(0, KC, pipe, 0)

    def sdrain(j, carry):
        pltpu.make_async_copy(msg0.at[j], ash0.at[idx_d.at[j]], ssem).wait()
        pltpu.make_async_copy(msg1.at[j], ash1.at[idx_d.at[j]], ssem).wait()
        return carry

    lax.fori_loop(0, KC, sdrain, 0)
    plsc.subcore_barrier()
    pltpu.sync_copy(ash0.at[pl.ds(s * RPN, RPN)],
                    aggp_hbm.at[pl.ds(c * NP2 + s * RPN, RPN)])
    pltpu.sync_copy(ash1.at[pl.ds(s * RPN, RPN)],
                    aggp_hbm.at[pl.ds(c * NP2 + NPAD + s * RPN, RPN)])


def _mm_body(x_ref, we_ref, fw_ref, wg_ref, cw_ref, u_ref):
    wc1 = jnp.dot(we_ref[...], fw_ref[...], preferred_element_type=jnp.float32)
    wc2 = jnp.dot(wg_ref[...], cw_ref[...], preferred_element_type=jnp.float32)
    u1 = jnp.dot(x_ref[...], wc1, preferred_element_type=jnp.float32)
    u2 = jnp.dot(x_ref[...], wc2, preferred_element_type=jnp.float32)
    pad = jnp.zeros((1, NPAD - N), jnp.float32)
    u_ref[...] = jnp.concatenate(
        [jnp.transpose(u1), pad, jnp.transpose(u2), pad], axis=1)


def _scale_body(u_ref, deg4_ref, v_ref):
    dout = deg4_ref[0:1, :] + deg4_ref[2:3, :]
    ns = jnp.where(dout > 0, lax.rsqrt(jnp.maximum(dout, 1.0)), 0.0)
    v_ref[...] = u_ref[...] * jnp.concatenate([ns, ns], axis=1)


def _fin_body(agg4_ref, deg4_ref, be_ref, fw_ref, fb_ref, bg_ref, cw_ref,
              cb_ref, s_ref, y_ref):
    din = deg4_ref[1:2, :] + deg4_ref[3:4, :]
    nd = jnp.where(din > 0, lax.rsqrt(jnp.maximum(din, 1.0)), 0.0)
    agg0 = agg4_ref[0:1, :] + agg4_ref[2:3, :]
    agg1 = agg4_ref[1:2, :] + agg4_ref[3:4, :]
    c1 = jnp.sum(be_ref[...] * fw_ref[...]) + fb_ref[0, 0]
    c2 = jnp.sum(bg_ref[...] * cw_ref[...]) + cb_ref[0, 0]
    s_ref[...] = agg0 * nd + c1
    y_ref[...] = agg1 * nd + c2


_mm_call = pl.pallas_call(
    _mm_body,
    out_shape=jax.ShapeDtypeStruct((1, NP2), jnp.float32),
)

_scale_call = pl.pallas_call(
    _scale_body,
    out_shape=jax.ShapeDtypeStruct((1, NP2), jnp.float32),
)

_fin_call = pl.pallas_call(
    _fin_body,
    out_shape=[
        jax.ShapeDtypeStruct((1, NPAD), jnp.float32),
        jax.ShapeDtypeStruct((1, NPAD), jnp.float32),
    ],
)


def kernel(x, edge_index, W_est, b_est, fc_w, fc_b, W_gnn, b_gnn, cls_w, cls_b):
    src = edge_index[0]
    dst = edge_index[1]
    pad = EPAD - E
    pad_idx = (N + (jnp.arange(pad, dtype=jnp.int32) % 32)).astype(jnp.int32)
    src0 = jnp.concatenate([src, pad_idx]).reshape(NW, KC, CH)
    dst0 = jnp.concatenate([dst, pad_idx]).reshape(NW, KC, CH)

    degp = _deg_kernel(src0, dst0)
    deg4 = degp.reshape(NC * 2, NPAD)
    u01 = _mm_call(x, W_est, fc_w, W_gnn, cls_w)
    v01 = _scale_call(u01, deg4)
    aggp = _agg_kernel(src0, dst0, v01.reshape(NP2))
    agg4 = aggp.reshape(NC * 2, NPAD)
    s_row, y_row = _fin_call(
        agg4, deg4,
        b_est.reshape(1, F), fc_w.reshape(1, F), fc_b.reshape(1, 1),
        b_gnn.reshape(1, F), cls_w.reshape(1, F), cls_b.reshape(1, 1),
    )
    return (y_row.reshape(NPAD, 1)[:N], s_row.reshape(NPAD, 1)[:N])


# speedup vs baseline: 86.5999x; 1.0829x over previous
import functools

import jax
import jax.numpy as jnp
from jax import lax
from jax.experimental import pallas as pl
from jax.experimental.pallas import tpu as pltpu
from jax.experimental.pallas import tpu_sc as plsc

N = 10000
E = 320000
F = 128
NC = 2
NS = 16
NW = NC * NS
CH = 128
KC = 80
EPT = CH * KC
EPAD = NW * EPT
NPAD = 10240
NP2 = 2 * NPAD
RPN = NPAD // NS
L = 16

_mesh = plsc.VectorSubcoreMesh(core_axis_name="c", subcore_axis_name="s",
                               num_cores=NC, num_subcores=NS)
_sc_params = pltpu.CompilerParams(use_tc_tiling_on_sc=False)


def _fill(ref, n, value):
    vec = jnp.full((L,), value, jnp.float32)
    for k in range(n // L):
        ref[pl.ds(k * L, L)] = vec


@functools.partial(
    pl.kernel,
    out_type=jax.ShapeDtypeStruct((NC * NP2,), jnp.float32),
    mesh=_mesh,
    compiler_params=_sc_params,
    scratch_types=[
        pltpu.VMEM((KC, CH), jnp.int32),
        pltpu.VMEM((KC, CH), jnp.int32),
        pltpu.VMEM((CH,), jnp.float32),
        pltpu.VMEM((RPN,), jnp.float32),
        pltpu.SemaphoreType.DMA,
        pltpu.VMEM_SHARED((NPAD,), jnp.float32),
        pltpu.VMEM_SHARED((NPAD,), jnp.float32),
    ],
)
def _deg_kernel(src0_hbm, dst0_hbm, degp_hbm,
                idx_s, idx_d, ones_v, zero_v, ssem, dsh_out, dsh_in):
    c = lax.axis_index("c")
    s = lax.axis_index("s")
    wid = c * NS + s
    _fill(ones_v, CH, 1.0)
    _fill(zero_v, RPN, 0.0)
    pltpu.sync_copy(zero_v, dsh_out.at[pl.ds(s * RPN, RPN)])
    pltpu.sync_copy(zero_v, dsh_in.at[pl.ds(s * RPN, RPN)])
    pltpu.sync_copy(src0_hbm.at[wid], idx_s)
    pltpu.sync_copy(dst0_hbm.at[wid], idx_d)
    plsc.subcore_barrier()

    def fire(j, carry):
        pltpu.async_copy(ones_v, dsh_out.at[idx_s.at[j]], ssem, add=True)
        pltpu.async_copy(ones_v, dsh_in.at[idx_d.at[j]], ssem, add=True)
        return carry

    lax.fori_loop(0, KC, fire, 0)

    def drain(j, carry):
        pltpu.make_async_copy(ones_v, dsh_out.at[idx_s.at[j]], ssem).wait()
        pltpu.make_async_copy(ones_v, dsh_in.at[idx_d.at[j]], ssem).wait()
        return carry

    lax.fori_loop(0, KC, drain, 0)
    plsc.subcore_barrier()
    pltpu.sync_copy(dsh_out.at[pl.ds(s * RPN, RPN)],
                    degp_hbm.at[pl.ds(c * NP2 + s * RPN, RPN)])
    pltpu.sync_copy(dsh_in.at[pl.ds(s * RPN, RPN)],
                    degp_hbm.at[pl.ds(c * NP2 + NPAD + s * RPN, RPN)])


@functools.partial(
    pl.kernel,
    out_type=jax.ShapeDtypeStruct((NC * NP2,), jnp.float32),
    mesh=_mesh,
    compiler_params=_sc_params,
    scratch_types=[
        pltpu.VMEM((KC, CH), jnp.int32),
        pltpu.VMEM((KC, CH), jnp.int32),
        pltpu.VMEM((KC, CH), jnp.float32),
        pltpu.VMEM((KC, CH), jnp.float32),
        pltpu.VMEM((RPN,), jnp.float32),
        pltpu.SemaphoreType.DMA,
        pltpu.SemaphoreType.DMA,
        pltpu.VMEM_SHARED((NPAD,), jnp.float32),
        pltpu.VMEM_SHARED((NPAD,), jnp.float32),
        pltpu.VMEM_SHARED((NPAD,), jnp.float32),
        pltpu.VMEM_SHARED((NPAD,), jnp.float32),
    ],
)
def _agg_kernel(src0_hbm, dst0_hbm, v_hbm, aggp_hbm,
                idx_s, idx_d, msg0, msg1, zero_v, gsem, ssem,
                ash0, ash1, vsh0, vsh1):
    c = lax.axis_index("c")
    s = lax.axis_index("s")
    wid = c * NS + s
    _fill(zero_v, RPN, 0.0)
    pltpu.sync_copy(zero_v, ash0.at[pl.ds(s * RPN, RPN)])
    pltpu.sync_copy(zero_v, ash1.at[pl.ds(s * RPN, RPN)])
    pltpu.sync_copy(v_hbm.at[pl.ds(s * RPN, RPN)],
                    vsh0.at[pl.ds(s * RPN, RPN)])
    pltpu.sync_copy(v_hbm.at[pl.ds(NPAD + s * RPN, RPN)],
                    vsh1.at[pl.ds(s * RPN, RPN)])
    pltpu.sync_copy(src0_hbm.at[wid], idx_s)
    pltpu.sync_copy(dst0_hbm.at[wid], idx_d)
    plsc.subcore_barrier()

    def gfire(j, carry):
        pltpu.async_copy(vsh0.at[idx_s.at[j]], msg0.at[j], gsem)
        pltpu.async_copy(vsh1.at[idx_s.at[j]], msg1.at[j], gsem)
        return carry

    lax.fori_loop(0, KC, gfire, 0)

    def pipe(j, carry):
        pltpu.make_async_copy(vsh0.at[idx_s.at[j]], msg0.at[j], gsem).wait()
        pltpu.make_async_copy(vsh1.at[idx_s.at[j]], msg1.at[j], gsem).wait()
        pltpu.async_copy(msg0.at[j], ash0.at[idx_d.at[j]], ssem, add=True)
        pltpu.async_copy(msg1.at[j], ash1.at[idx_d.at[j]], ssem, add=True)
        return carry

    lax.fori_loop(0, KC, pipe, 0)

    def sdrain(j, carry):
        pltpu.make_async_copy(msg0.at[j], ash0.at[idx_d.at[j]], ssem).wait()
        pltpu.make_async_copy(msg1.at[j], ash1.at[idx_d.at[j]], ssem).wait()
        return carry

    lax.fori_loop(0, KC, sdrain, 0)
    plsc.subcore_barrier()
    pltpu.sync_copy(ash0.at[pl.ds(s * RPN, RPN)],
                    aggp_hbm.at[pl.ds(c * NP2 + s * RPN, RPN)])
    pltpu.sync_copy(ash1.at[pl.ds(s * RPN, RPN)],
                    aggp_hbm.at[pl.ds(c * NP2 + NPAD + s * RPN, RPN)])


def _mm_body(x_ref, we_ref, fw_ref, wg_ref, cw_ref, u_ref):
    wc1 = jnp.dot(we_ref[...], fw_ref[...], preferred_element_type=jnp.float32)
    wc2 = jnp.dot(wg_ref[...], cw_ref[...], preferred_element_type=jnp.float32)
    u1 = jnp.dot(x_ref[...], wc1, preferred_element_type=jnp.float32)
    u2 = jnp.dot(x_ref[...], wc2, preferred_element_type=jnp.float32)
    pad = jnp.zeros((1, NPAD - N), jnp.float32)
    u_ref[...] = jnp.concatenate(
        [jnp.transpose(u1), pad, jnp.transpose(u2), pad], axis=1)


def _scale_body(u_ref, deg4_ref, v_ref):
    dout = deg4_ref[0:1, :] + deg4_ref[2:3, :]
    ns = jnp.where(dout > 0, lax.rsqrt(jnp.maximum(dout, 1.0)), 0.0)
    v_ref[...] = u_ref[...] * jnp.concatenate([ns, ns], axis=1)


def _fin_body(agg4_ref, deg4_ref, be_ref, fw_ref, fb_ref, bg_ref, cw_ref,
              cb_ref, s_ref, y_ref):
    din = deg4_ref[1:2, :] + deg4_ref[3:4, :]
    nd = jnp.where(din > 0, lax.rsqrt(jnp.maximum(din, 1.0)), 0.0)
    agg0 = agg4_ref[0:1, :] + agg4_ref[2:3, :]
    agg1 = agg4_ref[1:2, :] + agg4_ref[3:4, :]
    c1 = jnp.sum(be_ref[...] * fw_ref[...]) + fb_ref[0, 0]
    c2 = jnp.sum(bg_ref[...] * cw_ref[...]) + cb_ref[0, 0]
    s_ref[...] = agg0 * nd + c1
    y_ref[...] = agg1 * nd + c2


_mm_call = pl.pallas_call(
    _mm_body,
    out_shape=jax.ShapeDtypeStruct((1, NP2), jnp.float32),
)

_scale_call = pl.pallas_call(
    _scale_body,
    out_shape=jax.ShapeDtypeStruct((1, NP2), jnp.float32),
)

_fin_call = pl.pallas_call(
    _fin_body,
    out_shape=[
        jax.ShapeDtypeStruct((1, NPAD), jnp.float32),
        jax.ShapeDtypeStruct((1, NPAD), jnp.float32),
    ],
)


def kernel(x, edge_index, W_est, b_est, fc_w, fc_b, W_gnn, b_gnn, cls_w, cls_b):
    src = edge_index[0]
    dst = edge_index[1]
    ppw = EPT - E // NW
    pad_blk = jnp.broadcast_to(
        N + (jnp.arange(ppw, dtype=jnp.int32) % (NPAD - N)), (NW, ppw))
    src0 = jnp.concatenate([src.reshape(NW, E // NW), pad_blk],
                           axis=1).reshape(NW, KC, CH)
    dst0 = jnp.concatenate([dst.reshape(NW, E // NW), pad_blk],
                           axis=1).reshape(NW, KC, CH)

    degp = _deg_kernel(src0, dst0)
    deg4 = degp.reshape(NC * 2, NPAD)
    u01 = _mm_call(x, W_est, fc_w, W_gnn, cls_w)
    v01 = _scale_call(u01, deg4)
    aggp = _agg_kernel(src0, dst0, v01.reshape(NP2))
    agg4 = aggp.reshape(NC * 2, NPAD)
    s_row, y_row = _fin_call(
        agg4, deg4,
        b_est.reshape(1, F), fc_w.reshape(1, F), fc_b.reshape(1, 1),
        b_gnn.reshape(1, F), cls_w.reshape(1, F), cls_b.reshape(1, 1),
    )
    return (y_row.reshape(NPAD, 1)[:N], s_row.reshape(NPAD, 1)[:N])
